# Initial kernel scaffold; baseline (speedup 1.0000x reference)
import jax
import jax.numpy as jnp
from jax.experimental import pallas as pl


def kernel(Cls0, Shape0, Offset0, Cls1, Shape1, Offset1, Cls2, Shape2, Offset2):
    raise NotImplementedError("write your pallas kernel here")


# speedup vs baseline: 1.1420x; 1.1420x over previous
import functools

import jax
import jax.numpy as jnp
from jax import lax
from jax.experimental import pallas as pl
from jax.experimental.pallas import tpu as pltpu
from jax.experimental.pallas import tpu_sc as plsc

B = 16
TOPK = 60
THRESHOLD = 0.15
NMS_THRESHOLD = 0.05
NMS_TOPK = 20
CROP = 192.0
DIMS = (48, 24, 12)
NS = tuple(d * d * d for d in DIMS)
SHIFTS = (0, NS[0], NS[0] + NS[1])
CHUNKS = (NS[0] // 128, NS[1] // 128, 14)
NCAND = 3 * TOPK


def _topk_body(c0, c1, c2, scores_ref, idx_ref, s0, s1, s2, r0, r1, r2):
    ins = (c0, c1, c2)
    svs = (s0, s1, s2)
    rs = (r0, r1, r2)
    for lvl in range(3):
        s = jax.nn.sigmoid(ins[lvl][...])
        svs[lvl][...] = s
        rs[lvl][...] = jnp.max(s, axis=2)

    for lvl in range(3):
        sv = svs[lvl]
        rr = rs[lvl]
        ch = CHUNKS[lvl]
        shift = SHIFTS[lvl]
        ch_iota = lax.broadcasted_iota(jnp.int32, (1, ch), 1)
        lane_iota = lax.broadcasted_iota(jnp.int32, (1, 128), 1)

        def body(k, _, sv=sv, rr=rr, ch=ch, shift=shift, lvl=lvl,
                 ch_iota=ch_iota, lane_iota=lane_iota):
            for b in range(B):
                rrow = rr[b:b + 1, :]
                r = jnp.argmax(rrow, axis=1)[0]
                row = sv[b, pl.ds(r, 1), :]
                c = jnp.argmax(row, axis=1)[0]
                val = jnp.max(row)
                col = lvl * TOPK + k
                scores_ref[pl.ds(col, 1), b:b + 1] = jnp.reshape(val, (1, 1))
                gidx = shift + r * 128 + c
                idx_ref[pl.ds(col, 1), b:b + 1] = jnp.reshape(gidx, (1, 1))
                newrow = jnp.where(lane_iota == c, -1.0, row)
                sv[b, pl.ds(r, 1), :] = newrow
                rr[b:b + 1, :] = jnp.where(ch_iota == r, jnp.max(newrow), rrow)
            return 0

        lax.fori_loop(0, TOPK, body, 0)


def _run_topk(cls0, cls1, cls2):
    c0 = cls0.reshape(B, CHUNKS[0], 128)
    c1 = cls1.reshape(B, CHUNKS[1], 128)
    c2 = jnp.pad(cls2.reshape(B, NS[2]), ((0, 0), (0, 64)),
                 constant_values=-jnp.inf).reshape(B, CHUNKS[2], 128)
    scores_t, idx_t = pl.pallas_call(
        _topk_body,
        out_shape=[
            jax.ShapeDtypeStruct((NCAND, B), jnp.float32),
            jax.ShapeDtypeStruct((NCAND, B), jnp.int32),
        ],
        scratch_shapes=[
            pltpu.VMEM((B, CHUNKS[0], 128), jnp.float32),
            pltpu.VMEM((B, CHUNKS[1], 128), jnp.float32),
            pltpu.VMEM((B, CHUNKS[2], 128), jnp.float32),
            pltpu.VMEM((B, CHUNKS[0]), jnp.float32),
            pltpu.VMEM((B, CHUNKS[1]), jnp.float32),
            pltpu.VMEM((B, CHUNKS[2]), jnp.float32),
        ],
    )(c0, c1, c2)
    return scores_t.T, idx_t.T


_PER_W = 96
_NW = 32
_PADCAND = 1024


def _sc_gather_body(s0, o0, s1, o1, s2, o2, i0, i1, i2,
                    os0, oo0, os1, oo1, os2, oo2,
                    idx_v, rows_v, lanes_v, buf_s, buf_o, out_s, out_o,
                    sem_s, sem_o):
    wid = lax.axis_index("s") * 2 + lax.axis_index("c")
    tables = ((s0, o0, os0, oo0, i0, NS[0]),
              (s1, o1, os1, oo1, i1, NS[1]),
              (s2, o2, os2, oo2, i2, NS[2]))
    for (tab_s, tab_o, out_s_hbm, out_o_hbm, idx_hbm, n_lvl) in tables:
        pltpu.sync_copy(idx_hbm.at[pl.ds(wid * 32, 32)], idx_v)

        def v16(x):
            return jnp.full((16,), x, jnp.int32)

        for j in range(_PER_W // 16):
            e = v16(wid * _PER_W + j * 16) + lax.iota(jnp.int32, 16)
            q = (e * v16(21846)) >> v16(16)
            comp = e - q * v16(3)
            b = (q * v16(1093)) >> v16(16)
            b = jnp.where(b > v16(B - 1), v16(B - 1), b)
            n = plsc.load_gather(idx_v, [q - v16(wid * 32)])
            addr = (b * v16(3) + comp) * v16(n_lvl) + n
            rows_v[pl.ds(j * 16, 16)] = addr >> v16(7)
            lanes_v[pl.ds(j * 16, 16)] = addr & v16(127)
        cp_s = pltpu.async_copy(tab_s.at[rows_v], buf_s, sem_s)
        cp_o = pltpu.async_copy(tab_o.at[rows_v], buf_o, sem_o)
        cp_s.wait()
        cp_o.wait()
        for j in range(_PER_W // 16):
            ridx = v16(j * 16) + lax.iota(jnp.int32, 16)
            lidx = lanes_v[pl.ds(j * 16, 16)]
            out_s[pl.ds(j * 16, 16)] = plsc.load_gather(buf_s, [ridx, lidx])
            out_o[pl.ds(j * 16, 16)] = plsc.load_gather(buf_o, [ridx, lidx])
        pltpu.sync_copy(out_s, out_s_hbm.at[pl.ds(wid * _PER_W, _PER_W)])
        pltpu.sync_copy(out_o, out_o_hbm.at[pl.ds(wid * _PER_W, _PER_W)])


def _run_gather(idx, shape0, offset0, shape1, offset1, shape2, offset2):
    idx_lvls = []
    for lvl in range(3):
        loc = (idx[:, lvl * TOPK:(lvl + 1) * TOPK] - SHIFTS[lvl]).reshape(-1)
        idx_lvls.append(jnp.pad(loc, (0, _PADCAND - B * TOPK)))
    tabs = [a.reshape(-1, 128) for a in
            (shape0, offset0, shape1, offset1, shape2, offset2)]

    nel = _NW * _PER_W
    fn = pl.kernel(
        _sc_gather_body,
        out_type=[jax.ShapeDtypeStruct((nel,), jnp.float32)] * 6,
        mesh=plsc.VectorSubcoreMesh(core_axis_name="c", subcore_axis_name="s"),
        compiler_params=pltpu.CompilerParams(needs_layout_passes=False),
        scratch_types=[
            pltpu.VMEM((32,), jnp.int32),
            pltpu.VMEM((_PER_W,), jnp.int32),
            pltpu.VMEM((_PER_W,), jnp.int32),
            pltpu.VMEM((_PER_W, 128), jnp.float32),
            pltpu.VMEM((_PER_W, 128), jnp.float32),
            pltpu.VMEM((_PER_W,), jnp.float32),
            pltpu.VMEM((_PER_W,), jnp.float32),
            pltpu.SemaphoreType.DMA,
            pltpu.SemaphoreType.DMA,
        ],
    )
    os0, oo0, os1, oo1, os2, oo2 = fn(
        tabs[0], tabs[1], tabs[2], tabs[3], tabs[4], tabs[5],
        idx_lvls[0], idx_lvls[1], idx_lvls[2])
    shp = jnp.concatenate(
        [o[:B * TOPK * 3].reshape(B, TOPK, 3) for o in (os0, os1, os2)], axis=1)
    off = jnp.concatenate(
        [o[:B * TOPK * 3].reshape(B, TOPK, 3) for o in (oo0, oo1, oo2)], axis=1)
    return shp, off


def _nms_body(scores_ref, idx_ref, off_ref, shp_ref, out_ref, m_ref):
    n = NCAND
    s = scores_ref[...]
    idx = idx_ref[...]
    valid = s > THRESHOLD

    col = lax.broadcasted_iota(jnp.int32, (1, n), 1)
    dv = jnp.where(col < TOPK, DIMS[0], jnp.where(col < 2 * TOPK,
                                                  DIMS[1], DIMS[2]))
    shiftv = jnp.where(col < TOPK, SHIFTS[0], jnp.where(col < 2 * TOPK,
                                                        SHIFTS[1], SHIFTS[2]))
    stridev = (CROP / dv.astype(jnp.float32))
    nn = idx - shiftv
    dd = dv * dv
    z = nn // dd
    rem = nn - z * dd
    y = rem // dv
    x = rem - y * dv
    anchors = (z.astype(jnp.float32), y.astype(jnp.float32),
               x.astype(jnp.float32))
    centers = [(anchors[c] + off_ref[:, :, c]) * stridev for c in range(3)]
    shapes = [shp_ref[:, :, c] for c in range(3)]

    key = jnp.where(valid, -s, 2.0)
    ki = key[:, :, None]
    kj = key[:, None, :]
    ii = lax.broadcasted_iota(jnp.int32, (1, n, 1), 1)
    jj = lax.broadcasted_iota(jnp.int32, (1, 1, n), 2)
    rank = jnp.sum(((kj < ki) | ((kj == ki) & (jj < ii))).astype(jnp.float32),
                   axis=2).astype(jnp.int32)

    pp = lax.broadcasted_iota(jnp.int32, (1, 1, n), 2)
    oh_sort = (rank[:, :, None] == pp).astype(jnp.float32)

    def sort_q(q):
        return jnp.sum(oh_sort * q[:, :, None], axis=1)

    s_s = sort_q(s)
    v_s = sort_q(valid.astype(jnp.float32)) > 0.5
    cz, cy, cx = (sort_q(c) for c in centers)
    sz, sy, sx = (sort_q(c) for c in shapes)

    inter = None
    for (c, e) in ((cz, sz), (cy, sy), (cx, sx)):
        lo = c - e / 2.0
        hi = c + e / 2.0
        term = jnp.clip(jnp.minimum(hi[:, :, None], hi[:, None, :]) -
                        jnp.maximum(lo[:, :, None], lo[:, None, :]), 0.0, None)
        inter = term if inter is None else inter * term
    vol = sz * sy * sx
    m_ref[...] = inter / (vol[:, :, None] + vol[:, None, :] - inter + 1e-8)

    lane = lax.broadcasted_iota(jnp.int32, (1, n), 1)

    v_i32 = v_s.astype(jnp.int32)

    def nms_step(i, carry):
        supp, kept, cnt = carry
        row = m_ref[:, pl.ds(i, 1), :][:, 0, :]
        sel = lane == i
        supp_i = jnp.max(jnp.where(sel, supp, 0), axis=1, keepdims=True)
        val_i = jnp.max(jnp.where(sel, v_i32, 0), axis=1, keepdims=True)
        is_kept = (supp_i == 0) & (val_i > 0) & (cnt < NMS_TOPK)
        supp = jnp.where(is_kept & (row > NMS_THRESHOLD) & (lane > i),
                         1, supp)
        kept = jnp.where(is_kept & sel, 1, kept)
        cnt = cnt + is_kept.astype(jnp.int32)
        return supp, kept, cnt

    supp0 = jnp.zeros((B, n), dtype=jnp.int32)
    kept0 = jnp.zeros((B, n), dtype=jnp.int32)
    cnt0 = jnp.zeros((B, 1), dtype=jnp.int32)
    _, kept_i, _ = lax.fori_loop(0, n, nms_step, (supp0, kept0, cnt0))
    kept = kept_i > 0

    keptf = kept.astype(jnp.float32)
    pos = jnp.sum(keptf[:, None, :] * (jj < ii).astype(jnp.float32),
                  axis=2).astype(jnp.int32)
    target = jnp.where(kept, pos, 2 * n)

    oh_out = (target[:, :, None] == pp).astype(jnp.float32)

    def scat(q):
        return jnp.sum(oh_out * (q + 1.0)[:, :, None], axis=1) - 1.0

    cols = [scat(jnp.ones_like(s_s)), scat(s_s), scat(cz), scat(cy),
            scat(cx), scat(sz), scat(sy), scat(sx)]
    out_ref[...] = jnp.stack(cols, axis=-1)


def _run_nms(scores, idx, off, shp):
    return pl.pallas_call(
        _nms_body,
        out_shape=jax.ShapeDtypeStruct((B, NCAND, 8), jnp.float32),
        scratch_shapes=[pltpu.VMEM((B, NCAND, NCAND), jnp.float32)],
    )(scores, idx, off, shp)


def kernel(Cls0, Shape0, Offset0, Cls1, Shape1, Offset1, Cls2, Shape2,
           Offset2):
    scores, idx = _run_topk(Cls0.reshape(B, -1), Cls1.reshape(B, -1),
                            Cls2.reshape(B, -1))
    shp, off = _run_gather(idx, Shape0, Offset0, Shape1, Offset1,
                           Shape2, Offset2)
    return _run_nms(scores, idx, off, shp)
